# Initial kernel scaffold; baseline (speedup 1.0000x reference)
import jax
import jax.numpy as jnp
from jax.experimental import pallas as pl


def kernel(x, pos_table):
    raise NotImplementedError("write your pallas kernel here")


# speedup vs baseline: 1.3983x; 1.3983x over previous
import jax
import jax.numpy as jnp
from jax.experimental import pallas as pl
from jax.experimental.pallas import tpu as pltpu

MAXLEN_ROWS = 2048
DIM = 1024
ROW_BLOCK = 256


def _add_body(x_ref, pos_ref, out_ref):
    out_ref[...] = x_ref[...] + pos_ref[...]


def kernel(x, pos_table):
    batch, maxlen, dim = x.shape
    grid = (batch, maxlen // ROW_BLOCK)
    return pl.pallas_call(
        _add_body,
        grid=grid,
        in_specs=[
            pl.BlockSpec((1, ROW_BLOCK, dim), lambda b, i: (b, i, 0)),
            pl.BlockSpec((ROW_BLOCK, dim), lambda b, i: (i, 0)),
        ],
        out_specs=pl.BlockSpec((1, ROW_BLOCK, dim), lambda b, i: (b, i, 0)),
        out_shape=jax.ShapeDtypeStruct(x.shape, x.dtype),
    )(x, pos_table)
